# Initial kernel scaffold; baseline (speedup 1.0000x reference)
import jax
import jax.numpy as jnp
from jax.experimental import pallas as pl


def kernel(xyz_q, lat_rep, xyz, points, W_delta1, b_delta1, W_delta2, b_delta2, W_gamma1, b_gamma1, W_gamma2, b_gamma2, W_kg, W_vg, W_qs, W_ks, W_vs):
    raise NotImplementedError("write your pallas kernel here")


# speedup vs baseline: 17.1853x; 17.1853x over previous
import jax
import jax.numpy as jnp
from jax.experimental import pallas as pl
from jax.experimental.pallas import tpu as pltpu

B, NQ, N, DIM_G, DIM_INP, DIM, K = 4, 1024, 1024, 256, 128, 256, 16
MQ = 128
TW = 144
BIG = 3.0e38


def _fused(xyzq_ref, xyzT_ref, table_ref, lat_ref,
           wd1_ref, bd1_ref, wd2_ref, bd2_ref,
           wg1_ref, bg1_ref, wg2_ref, bg2_ref,
           wkg_ref, wvg_ref, wqs_ref, wkv_ref,
           out_ref):
    f32 = jnp.float32
    xq = xyzq_ref[0]
    xt = xyzT_ref[0]

    d = jnp.zeros((MQ, N), f32)
    for j in range(3):
        t = xq[:, j:j + 1] - xt[j:j + 1, :]
        d = d + t * t

    tbl = table_ref[0]
    iota = jax.lax.broadcasted_iota(jnp.int32, (MQ, N), 1)
    gs = []
    for _ in range(K):
        m = jnp.min(d, axis=1, keepdims=True)
        idx = jnp.min(jnp.where(d == m, iota, N), axis=1, keepdims=True)
        sel = iota == idx
        d = jnp.where(sel, BIG, d)
        oh = sel.astype(f32)
        gs.append(jax.lax.dot_general(
            oh, tbl, (((1,), (0,)), ((), ())), preferred_element_type=f32))
    gath = jnp.concatenate([g[None] for g in gs], axis=0)

    R = K * MQ
    g2 = gath.reshape(R, TW)
    gp = g2[:, :DIM_INP]
    gx = g2[:, DIM_INP:DIM_INP + 3]

    kv = jnp.dot(gp, wkv_ref[...], preferred_element_type=f32)
    kloc = kv[:, :DIM]
    vloc = kv[:, DIM:]

    qxb = jnp.broadcast_to(xq[None], (K, MQ, 3)).reshape(R, 3)
    dv = qxb - gx
    h = jnp.maximum(
        jnp.dot(dv, wd1_ref[...], preferred_element_type=f32) + bd1_ref[...], 0.0)
    pos = jnp.dot(h, wd2_ref[...], preferred_element_type=f32) + bd2_ref[...]

    lr = lat_ref[0]
    qg = jnp.dot(lr, wqs_ref[...], preferred_element_type=f32)
    kg = jnp.dot(lr, wkg_ref[...], preferred_element_type=f32)
    vg = jnp.dot(lr, wvg_ref[...], preferred_element_type=f32)

    ain = qg - kloc + pos
    t1 = jnp.maximum(
        jnp.dot(ain, wg1_ref[...], preferred_element_type=f32) + bg1_ref[...], 0.0)
    a = jnp.dot(t1, wg2_ref[...], preferred_element_type=f32) + bg2_ref[...]

    gt = jnp.maximum(
        jnp.dot(qg - kg, wg1_ref[...], preferred_element_type=f32) + bg1_ref[...], 0.0)
    glog = jnp.dot(gt, wg2_ref[...], preferred_element_type=f32) + bg2_ref[...]

    a3 = a.reshape(K, MQ, DIM)
    vpp3 = (vloc + pos).reshape(K, MQ, DIM)
    m = jnp.maximum(jnp.max(a3, axis=0), glog)
    e3 = jnp.exp(a3 - m[None])
    s = jnp.sum(e3, axis=0)
    num = jnp.sum(e3 * vpp3, axis=0)
    eg = jnp.exp(glog - m)
    s = s + eg
    num = num + eg * vg
    out_ref[0] = num / s


def kernel(xyz_q, lat_rep, xyz, points, W_delta1, b_delta1, W_delta2, b_delta2,
           W_gamma1, b_gamma1, W_gamma2, b_gamma2, W_kg, W_vg, W_qs, W_ks, W_vs):
    xyzT = jnp.swapaxes(xyz, 1, 2)
    pad = jnp.zeros((B, N, TW - DIM_INP - 3), jnp.float32)
    table = jnp.concatenate([points, xyz, pad], axis=-1)
    wkv = jnp.concatenate([W_ks.T, W_vs.T], axis=1)

    full = lambda shape: pl.BlockSpec(shape, lambda b, q: tuple(0 for _ in shape))
    row = lambda: pl.BlockSpec((1, DIM), lambda b, q: (0, 0))

    out = pl.pallas_call(
        _fused,
        grid=(B, NQ // MQ),
        in_specs=[
            pl.BlockSpec((1, MQ, 3), lambda b, q: (b, q, 0)),
            pl.BlockSpec((1, 3, N), lambda b, q: (b, 0, 0)),
            pl.BlockSpec((1, N, TW), lambda b, q: (b, 0, 0)),
            pl.BlockSpec((1, 1, DIM_G), lambda b, q: (b, 0, 0)),
            full((3, DIM)),
            row(),
            full((DIM, DIM)),
            row(),
            full((DIM, DIM)),
            row(),
            full((DIM, DIM)),
            row(),
            full((DIM_G, DIM)),
            full((DIM_G, DIM)),
            full((DIM_G, DIM)),
            full((DIM_INP, 2 * DIM)),
        ],
        out_specs=pl.BlockSpec((1, MQ, DIM), lambda b, q: (b, q, 0)),
        out_shape=jax.ShapeDtypeStruct((B, NQ, DIM), jnp.float32),
    )(xyz_q, xyzT, table, lat_rep.reshape(B, 1, DIM_G),
      W_delta1.T, b_delta1.reshape(1, DIM),
      W_delta2.T, b_delta2.reshape(1, DIM),
      W_gamma1.T, b_gamma1.reshape(1, DIM),
      W_gamma2.T, b_gamma2.reshape(1, DIM),
      W_kg.T, W_vg.T, W_qs.T, wkv)
    return out


# speedup vs baseline: 20.0588x; 1.1672x over previous
import jax
import jax.numpy as jnp
from jax.experimental import pallas as pl
from jax.experimental.pallas import tpu as pltpu

B, NQ, N, DIM_G, DIM_INP, DIM, K = 4, 1024, 1024, 256, 128, 256, 16
MQ = 128
TW = 144
BIG = 3.0e38


def _fused(xyzq_ref, xyzT_ref, table_ref, lat_ref,
           wd1_ref, bd1_ref, wd2_ref, bd2_ref,
           wg1_ref, bg1_ref, wg2_ref, bg2_ref,
           wkg_ref, wvg_ref, wqs_ref, wkv_ref,
           out_ref):
    f32 = jnp.float32
    xq = xyzq_ref[0]
    xt = xyzT_ref[0]

    d = jnp.zeros((MQ, N), f32)
    for j in range(3):
        t = xq[:, j:j + 1] - xt[j:j + 1, :]
        d = d + t * t

    tbl = table_ref[0]
    iota = jax.lax.broadcasted_iota(jnp.int32, (MQ, N), 1).astype(f32)
    gs = []
    for _ in range(K):
        m = jnp.min(d, axis=1, keepdims=True)
        idx = jnp.min(jnp.where(d == m, iota, float(N)), axis=1, keepdims=True)
        sel = iota == idx
        d = jnp.where(sel, BIG, d)
        oh = sel.astype(f32)
        gs.append(jax.lax.dot_general(
            oh, tbl, (((1,), (0,)), ((), ())), preferred_element_type=f32))
    gath = jnp.concatenate([g[None] for g in gs], axis=0)

    R = K * MQ
    g2 = gath.reshape(R, TW)
    gp = g2[:, :DIM_INP]
    gx = g2[:, DIM_INP:DIM_INP + 3]

    bf16 = jnp.bfloat16

    kv = jnp.dot(gp.astype(bf16), wkv_ref[...].astype(bf16),
                 preferred_element_type=f32)
    kloc = kv[:, :DIM]
    vloc = kv[:, DIM:]

    qxb = jnp.broadcast_to(xq[None], (K, MQ, 3)).reshape(R, 3)
    dv = qxb - gx
    h = jnp.maximum(
        jnp.dot(dv, wd1_ref[...], preferred_element_type=f32) + bd1_ref[...], 0.0)
    pos = jnp.dot(h.astype(bf16), wd2_ref[...].astype(bf16),
                  preferred_element_type=f32) + bd2_ref[...]

    lr = lat_ref[0]
    qg = jnp.dot(lr, wqs_ref[...], preferred_element_type=f32)
    kg = jnp.dot(lr, wkg_ref[...], preferred_element_type=f32)
    vg = jnp.dot(lr, wvg_ref[...], preferred_element_type=f32)

    ain = qg - kloc + pos
    t1 = jnp.maximum(
        jnp.dot(ain.astype(bf16), wg1_ref[...].astype(bf16),
                preferred_element_type=f32) + bg1_ref[...], 0.0)
    a = jnp.dot(t1.astype(bf16), wg2_ref[...].astype(bf16),
                preferred_element_type=f32) + bg2_ref[...]

    gt = jnp.maximum(
        jnp.dot(qg - kg, wg1_ref[...], preferred_element_type=f32) + bg1_ref[...], 0.0)
    glog = jnp.dot(gt, wg2_ref[...], preferred_element_type=f32) + bg2_ref[...]

    a3 = a.reshape(K, MQ, DIM)
    vpp3 = (vloc + pos).reshape(K, MQ, DIM)
    m = jnp.maximum(jnp.max(a3, axis=0), glog)
    e3 = jnp.exp(a3 - m[None])
    s = jnp.sum(e3, axis=0)
    num = jnp.sum(e3 * vpp3, axis=0)
    eg = jnp.exp(glog - m)
    s = s + eg
    num = num + eg * vg
    out_ref[0] = num / s


def kernel(xyz_q, lat_rep, xyz, points, W_delta1, b_delta1, W_delta2, b_delta2,
           W_gamma1, b_gamma1, W_gamma2, b_gamma2, W_kg, W_vg, W_qs, W_ks, W_vs):
    xyzT = jnp.swapaxes(xyz, 1, 2)
    pad = jnp.zeros((B, N, TW - DIM_INP - 3), jnp.float32)
    table = jnp.concatenate([points, xyz, pad], axis=-1)
    wkv = jnp.concatenate([W_ks.T, W_vs.T], axis=1)

    full = lambda shape: pl.BlockSpec(shape, lambda b, q: tuple(0 for _ in shape))
    row = lambda: pl.BlockSpec((1, DIM), lambda b, q: (0, 0))

    out = pl.pallas_call(
        _fused,
        grid=(B, NQ // MQ),
        in_specs=[
            pl.BlockSpec((1, MQ, 3), lambda b, q: (b, q, 0)),
            pl.BlockSpec((1, 3, N), lambda b, q: (b, 0, 0)),
            pl.BlockSpec((1, N, TW), lambda b, q: (b, 0, 0)),
            pl.BlockSpec((1, 1, DIM_G), lambda b, q: (b, 0, 0)),
            full((3, DIM)),
            row(),
            full((DIM, DIM)),
            row(),
            full((DIM, DIM)),
            row(),
            full((DIM, DIM)),
            row(),
            full((DIM_G, DIM)),
            full((DIM_G, DIM)),
            full((DIM_G, DIM)),
            full((DIM_INP, 2 * DIM)),
        ],
        out_specs=pl.BlockSpec((1, MQ, DIM), lambda b, q: (b, q, 0)),
        out_shape=jax.ShapeDtypeStruct((B, NQ, DIM), jnp.float32),
    )(xyz_q, xyzT, table, lat_rep.reshape(B, 1, DIM_G),
      W_delta1.T, b_delta1.reshape(1, DIM),
      W_delta2.T, b_delta2.reshape(1, DIM),
      W_gamma1.T, b_gamma1.reshape(1, DIM),
      W_gamma2.T, b_gamma2.reshape(1, DIM),
      W_kg.T, W_vg.T, W_qs.T, wkv)
    return out
